# baseline (speedup 1.0000x reference)
import functools

import jax
import jax.numpy as jnp
from jax import lax
from jax.experimental import pallas as pl
from jax.experimental.pallas import tpu as pltpu
from jax.experimental.pallas import tpu_sc as plsc

BATCH = 4096
SEQ = 200
DIM = 64
TOTAL = BATCH * SEQ

NUM_CORES = 2
NUM_SUBCORES = 16
NW = NUM_CORES * NUM_SUBCORES
B_PER_W = TOTAL // NW
CHUNK = 1024
N_CHUNKS = B_PER_W // CHUNK

_MESH = plsc.VectorSubcoreMesh(core_axis_name="c", subcore_axis_name="s")


@functools.partial(
    pl.kernel,
    mesh=_MESH,
    out_type=jax.ShapeDtypeStruct((TOTAL, DIM), jnp.float32),
    scratch_types=[
        pltpu.VMEM((CHUNK,), jnp.int32),
        pltpu.VMEM((CHUNK, DIM), jnp.float32),
        pltpu.SemaphoreType.DMA,
    ],
    compiler_params=pltpu.CompilerParams(use_tc_tiling_on_sc=False),
)
def _gather_rows(idx_hbm, table_hbm, out_hbm, idx_v, rows_v, sem):
    wid = lax.axis_index("s") * NUM_CORES + lax.axis_index("c")
    base = wid * B_PER_W

    def body(i, carry):
        off = base + i * CHUNK
        pltpu.sync_copy(idx_hbm.at[pl.ds(off, CHUNK)], idx_v)
        pltpu.async_copy(table_hbm.at[idx_v], rows_v, sem).wait()
        pltpu.sync_copy(rows_v, out_hbm.at[pl.ds(off, CHUNK)])
        return carry

    lax.fori_loop(0, N_CHUNKS, body, 0)


def kernel(token_seqs, emb_table):
    idx = token_seqs.reshape(-1).astype(jnp.int32)
    out = _gather_rows(idx, emb_table)
    return out.reshape(BATCH, SEQ, DIM)


# speedup vs baseline: 1.0201x; 1.0201x over previous
import functools

import jax
import jax.numpy as jnp
from jax import lax
from jax.experimental import pallas as pl
from jax.experimental.pallas import tpu as pltpu
from jax.experimental.pallas import tpu_sc as plsc

BATCH = 4096
SEQ = 200
DIM = 64
TOTAL = BATCH * SEQ

NUM_CORES = 2
NUM_SUBCORES = 16
NW = NUM_CORES * NUM_SUBCORES
B_PER_W = TOTAL // NW

NBUF = 4
CHUNK = 400
N_CHUNKS = B_PER_W // CHUNK
LOOKAHEAD = 2
N_OUTER = N_CHUNKS // NBUF

_MESH = plsc.VectorSubcoreMesh(core_axis_name="c", subcore_axis_name="s")

_SCRATCH = (
    [pltpu.VMEM((CHUNK,), jnp.int32) for _ in range(NBUF)]
    + [pltpu.VMEM((CHUNK, DIM), jnp.float32) for _ in range(NBUF)]
    + [pltpu.SemaphoreType.DMA for _ in range(2 * NBUF)]
)


@functools.partial(
    pl.kernel,
    mesh=_MESH,
    out_type=jax.ShapeDtypeStruct((TOTAL, DIM), jnp.float32),
    scratch_types=_SCRATCH,
    compiler_params=pltpu.CompilerParams(use_tc_tiling_on_sc=False),
)
def _gather_rows(idx_hbm, table_hbm, out_hbm, *refs):
    idx_v = refs[0:NBUF]
    rows_v = refs[NBUF : 2 * NBUF]
    sg = refs[2 * NBUF : 3 * NBUF]
    ss = refs[3 * NBUF : 4 * NBUF]

    wid = lax.axis_index("s") * NUM_CORES + lax.axis_index("c")
    base = wid * B_PER_W

    def launch(i, b):
        off = base + i * CHUNK
        pltpu.sync_copy(idx_hbm.at[pl.ds(off, CHUNK)], idx_v[b])
        pltpu.async_copy(table_hbm.at[idx_v[b]], rows_v[b], sg[b])

    def gather_wait(b):
        pltpu.make_async_copy(table_hbm.at[idx_v[b]], rows_v[b], sg[b]).wait()

    def store_start(i, b):
        off = base + i * CHUNK
        pltpu.async_copy(rows_v[b], out_hbm.at[pl.ds(off, CHUNK)], ss[b])

    def store_wait(b):
        pltpu.make_async_copy(
            rows_v[b], out_hbm.at[pl.ds(base, CHUNK)], ss[b]
        ).wait()

    for i in range(LOOKAHEAD):
        launch(i, i % NBUF)

    for b in range(NBUF):
        gather_wait(b)
        store_start(b, b)
        ni = b + LOOKAHEAD
        nb = ni % NBUF
        if ni < NBUF:
            launch(ni, nb)
        else:
            store_wait(nb)
            launch(ni, nb)

    def outer(g, carry):
        for b in range(NBUF):
            i = g * NBUF + b
            gather_wait(b)
            store_start(i, b)
            ni = i + LOOKAHEAD
            nb = (b + LOOKAHEAD) % NBUF

            @pl.when(ni < N_CHUNKS)
            def _():
                store_wait(nb)
                launch(ni, nb)

        return carry

    lax.fori_loop(1, N_OUTER, outer, 0)

    for b in range(NBUF):
        store_wait(b)


def kernel(token_seqs, emb_table):
    idx = token_seqs.reshape(-1).astype(jnp.int32)
    out = _gather_rows(idx, emb_table)
    return out.reshape(BATCH, SEQ, DIM)
